# baseline (speedup 1.0000x reference)
import functools
import math

import jax
import jax.numpy as jnp
from jax import lax
from jax.experimental import pallas as pl
from jax.experimental.pallas import tpu as pltpu
from jax.experimental.pallas import tpu_sc as plsc

D_H = 64
BBLK = 128
NUM_CORES = 2
NUM_SUBCORES = 16
NW = NUM_CORES * NUM_SUBCORES


def _emb_body(seq_len, n_batch, src_hbm, pe_hbm, table_hbm, out_hbm,
              idxs, rows_v, obuf, pe_v, gsem0, gsem1, ssem0, ssem1):
    gsem = (gsem0, gsem1)
    ssem = (ssem0, ssem1)
    wid = lax.axis_index("s") * NUM_CORES + lax.axis_index("c")
    b0 = wid * BBLK

    pltpu.sync_copy(src_hbm.at[:, pl.ds(b0, BBLK)], idxs)
    pltpu.sync_copy(pe_hbm, pe_v)

    def fire_gather(l, b):
        pltpu.async_copy(table_hbm.at[idxs.at[l]], rows_v.at[b], gsem[b])

    def wait_gather(b):
        pltpu.make_async_copy(table_hbm.at[idxs.at[0]], rows_v.at[b],
                              gsem[b]).wait()

    def fire_store(l, b):
        pltpu.async_copy(obuf.at[b], out_hbm.at[l, :, pl.ds(b0, BBLK)],
                         ssem[b])

    def wait_store(b):
        pltpu.make_async_copy(obuf.at[b], out_hbm.at[0, :, pl.ds(b0, BBLK)],
                              ssem[b]).wait()

    def compute(l, b):
        @pl.loop(0, D_H // 16)
        def _(cg):
            pv16 = pe_v[l, pl.ds(cg * 16, 16)]
            for cc in range(16):
                c = cg * 16 + cc
                p = jnp.broadcast_to(pv16[cc], (16,))
                for g in range(BBLK // 16):
                    ridx = g * 16 + lax.iota(jnp.int32, 16)
                    cidx = jnp.broadcast_to(jnp.int32(0), (16,)) + c
                    v = plsc.load_gather(rows_v.at[b], [ridx, cidx])
                    obuf[b, c, pl.ds(g * 16, 16)] = v * 8.0 + p

    def l_step(l, b, nb):
        @pl.when(l + 1 < seq_len)
        def _():
            fire_gather(l + 1, nb)

        wait_gather(b)

        @pl.when(l >= 2)
        def _():
            wait_store(b)

        compute(l, b)
        fire_store(l, b)

    fire_gather(0, 0)

    @pl.loop(0, seq_len // 2)
    def _(l2):
        l_step(l2 * 2, 0, 1)
        l_step(l2 * 2 + 1, 1, 0)

    wait_store(0)
    wait_store(1)


def _build_sc_call(n_batch, seq_len, n_vocab):
    mesh = plsc.VectorSubcoreMesh(core_axis_name="c", subcore_axis_name="s")
    return functools.partial(
        pl.kernel,
        out_type=jax.ShapeDtypeStruct((seq_len, D_H, n_batch), jnp.float32),
        mesh=mesh,
        scratch_types=[
            pltpu.VMEM((seq_len, BBLK), jnp.int32),
            pltpu.VMEM((2, BBLK, 2 * D_H), jnp.float32),
            pltpu.VMEM((2, D_H, BBLK), jnp.float32),
            pltpu.VMEM((seq_len, D_H), jnp.float32),
            pltpu.SemaphoreType.DMA,
            pltpu.SemaphoreType.DMA,
            pltpu.SemaphoreType.DMA,
            pltpu.SemaphoreType.DMA,
        ],
        compiler_params=pltpu.CompilerParams(use_tc_tiling_on_sc=False,
                                             needs_layout_passes=False),
    )(functools.partial(_emb_body, seq_len, n_batch))


def kernel(src, table, pe):
    b, l = src.shape
    assert b % (NW * BBLK) == 0 or b == NW * BBLK
    srcT = src.T
    tpad = jnp.concatenate(
        [table, jnp.zeros((table.shape[0], 2 * D_H - D_H), table.dtype)],
        axis=1)
    pe_seq = pe[0, :l, :]
    out2 = _build_sc_call(b, l, table.shape[0])(srcT, pe_seq, tpad)
    return jnp.transpose(out2, (2, 0, 1))
